# baseline (speedup 1.0000x reference)
import functools

import jax
import jax.numpy as jnp
from jax import lax
from jax.experimental import pallas as pl
from jax.experimental.pallas import tpu as pltpu
from jax.experimental.pallas import tpu_sc as plsc

N, C, L = 1024, 80, 256
R = N * C
NUM_CORES = 2
NUM_SUBCORES = 16
W = NUM_CORES * NUM_SUBCORES
RPW = R // W
CH = 64
CPAD = 128
LANE = 16


def _sc_body(s_hbm, t_hbm, tgt_hbm, out_hbm,
             tgt_v, idx_v, sbuf, tbuf, acc_v, sem_s, sem_t):
    wid = lax.axis_index("s") * NUM_CORES + lax.axis_index("c")
    base = wid * RPW
    lanes = lax.iota(jnp.int32, LANE)
    zero16 = jnp.zeros((LANE,), jnp.float32)

    pltpu.sync_copy(tgt_hbm.at[pl.ds(base, RPW)], tgt_v)

    def zbody(i, carry):
        acc_v[pl.ds(i * LANE, LANE)] = zero16
        return carry
    lax.fori_loop(0, CPAD // LANE, zbody, 0)

    def cbody(i, wptr):
        tv = tgt_v[pl.ds(i * LANE, LANE)]
        m = tv > 0
        mi = m.astype(jnp.int32)
        pos = wptr + plsc.cumsum(mi) - 1
        rowid = base + i * LANE + lanes
        plsc.store_scatter(idx_v, [pos], rowid, mask=m)
        return wptr + jnp.sum(mi)
    count = lax.fori_loop(0, RPW // LANE, cbody, jnp.int32(0))

    basevec = jnp.full((LANE,), base, jnp.int32)
    for t in range(CH // LANE):
        plsc.store_scatter(idx_v, [count + t * LANE + lanes], basevec)

    nchunks = (count + CH - 1) // CH

    def chunk_body(g, carry):
        idxsl = idx_v.at[pl.ds(g * CH, CH)]
        cps = pltpu.async_copy(s_hbm.at[idxsl], sbuf, sem_s)
        cpt = pltpu.async_copy(t_hbm.at[idxsl], tbuf, sem_t)
        cps.wait()
        cpt.wait()
        rows_here = jnp.minimum(CH, count - g * CH)

        def row_body(j, carry2):
            a0 = zero16
            a1 = zero16
            a2 = zero16
            a3 = zero16
            accs = [a0, a1, a2, a3]
            for k in range(L // LANE):
                sv = sbuf[j, pl.ds(k * LANE, LANE)]
                tv2 = tbuf[j, pl.ds(k * LANE, LANE)]
                d = sv - tv2
                ad = jnp.abs(d)
                h = jnp.where(ad < 1.0, 0.5 * d * d, ad - 0.5)
                accs[k % 4] = accs[k % 4] + h
            hs = jnp.sum((accs[0] + accs[1]) + (accs[2] + accs[3]))
            rid_vec = idx_v[pl.ds(g * CH + j, LANE)]
            cls_vec = lax.rem(rid_vec, jnp.int32(C))
            plsc.addupdate_scatter(acc_v, [cls_vec], jnp.full((LANE,), hs),
                                   mask=lanes == 0)
            return carry2
        lax.fori_loop(0, rows_here, row_body, 0)
        return carry
    lax.fori_loop(0, nchunks, chunk_body, 0)

    pltpu.sync_copy(acc_v, out_hbm.at[wid])


@functools.partial(
    pl.kernel,
    out_type=jax.ShapeDtypeStruct((W, CPAD), jnp.float32),
    mesh=plsc.VectorSubcoreMesh(core_axis_name="c", subcore_axis_name="s"),
    compiler_params=pltpu.CompilerParams(needs_layout_passes=False),
    scratch_types=[
        pltpu.VMEM((RPW,), jnp.int32),
        pltpu.VMEM((RPW + CH,), jnp.int32),
        pltpu.VMEM((CH, L), jnp.float32),
        pltpu.VMEM((CH, L), jnp.float32),
        pltpu.VMEM((CPAD,), jnp.float32),
        pltpu.SemaphoreType.DMA,
        pltpu.SemaphoreType.DMA,
    ],
)
def _sc_partial_sums(s_hbm, t_hbm, tgt_hbm, out_hbm, *rest):
    _sc_body(s_hbm, t_hbm, tgt_hbm, out_hbm, *rest)


def _fin_body(parts_ref, tgt_ref, out_ref):
    s = jnp.sum(parts_ref[...], axis=0, keepdims=True)
    npos = jnp.sum(tgt_ref[...], axis=0, keepdims=True)
    denom = jnp.maximum(npos * jnp.float32(L), 1.0)
    valid = (npos > 1.0).astype(jnp.float32)
    out_ref[0, 0] = jnp.sum(s / denom * valid)


def kernel(le_student, le_teacher, targets):
    s2 = le_student.reshape(R, L)
    t2 = le_teacher.reshape(R, L)
    tgt = targets.reshape(R)
    parts = _sc_partial_sums(s2, t2, tgt)
    tgt_f = jnp.pad(targets.astype(jnp.float32), ((0, 0), (0, CPAD - C)))
    loss = pl.pallas_call(
        _fin_body,
        out_shape=jax.ShapeDtypeStruct((1, 1), jnp.float32),
        out_specs=pl.BlockSpec(memory_space=pltpu.SMEM),
    )(parts, tgt_f)
    return loss[0, 0]


# speedup vs baseline: 1.4575x; 1.4575x over previous
import functools

import jax
import jax.numpy as jnp
from jax import lax
from jax.experimental import pallas as pl
from jax.experimental.pallas import tpu as pltpu
from jax.experimental.pallas import tpu_sc as plsc

N, C, L = 1024, 80, 256
R = N * C
NUM_CORES = 2
NUM_SUBCORES = 16
W = NUM_CORES * NUM_SUBCORES
RPW = R // W
CH = 64
CPAD = 128
ACC = 2 * CPAD
LANE = 16


def _sc_body(s_hbm, t_hbm, tgt_hbm, out_hbm,
             tgt_v, idx_v, sbuf0, sbuf1, tbuf0, tbuf1, rs_v, acc_v,
             sem_s0, sem_s1, sem_t0, sem_t1):
    wid = lax.axis_index("s") * NUM_CORES + lax.axis_index("c")
    base = wid * RPW
    lanes = lax.iota(jnp.int32, LANE)
    zero16 = jnp.zeros((LANE,), jnp.float32)

    pltpu.sync_copy(tgt_hbm.at[pl.ds(base, RPW)], tgt_v)

    def zbody(i, carry):
        acc_v[pl.ds(i * LANE, LANE)] = zero16
        return carry
    lax.fori_loop(0, ACC // LANE, zbody, 0)

    def cbody(i, wptr):
        tv = tgt_v[pl.ds(i * LANE, LANE)]
        m = tv > 0
        mi = m.astype(jnp.int32)
        pos = wptr + plsc.cumsum(mi) - 1
        rowid = base + i * LANE + lanes
        plsc.store_scatter(idx_v, [pos], rowid, mask=m)
        cls = lax.rem(rowid, jnp.int32(C))
        plsc.addupdate_scatter(acc_v, [cls + CPAD], m.astype(jnp.float32))
        return wptr + jnp.sum(mi)
    count = lax.fori_loop(0, RPW // LANE, cbody, jnp.int32(0))

    basevec = jnp.full((LANE,), base, jnp.int32)
    for t in range(CH // LANE):
        plsc.store_scatter(idx_v, [count + t * LANE + lanes], basevec)

    nchunks = (count + CH - 1) // CH
    sbufs = (sbuf0, sbuf1)
    tbufs = (tbuf0, tbuf1)
    sems_s = (sem_s0, sem_s1)
    sems_t = (sem_t0, sem_t1)

    def fire(g, b):
        idxsl = idx_v.at[pl.ds(g * CH, CH)]
        pltpu.async_copy(s_hbm.at[idxsl], sbufs[b], sems_s[b])
        pltpu.async_copy(t_hbm.at[idxsl], tbufs[b], sems_t[b])

    def drain(b):
        dummy = s_hbm.at[pl.ds(0, CH)]
        pltpu.make_async_copy(dummy, sbufs[b], sems_s[b]).wait()
        pltpu.make_async_copy(dummy, tbufs[b], sems_t[b]).wait()

    def compute_chunk(g, b):
        sbuf = sbufs[b]
        tbuf = tbufs[b]

        def group_body(gg, carry):
            off = g * CH + gg * LANE
            rid16 = idx_v[pl.ds(off, LANE)]
            cls16 = lax.rem(rid16, jnp.int32(C))
            valid16 = (off + lanes) < count
            for j in range(LANE):
                rbase = gg * LANE + j
                accs = [zero16, zero16, zero16, zero16]
                for k in range(L // LANE):
                    sv = sbuf[rbase, pl.ds(k * LANE, LANE)]
                    tv2 = tbuf[rbase, pl.ds(k * LANE, LANE)]
                    d = sv - tv2
                    a = jnp.abs(d)
                    m = jnp.minimum(a, 1.0)
                    accs[k % 4] = accs[k % 4] + m * (a - 0.5 * m)
                rs_v[pl.ds(j * LANE, LANE)] = (accs[0] + accs[1]) + (accs[2] + accs[3])
            t0 = zero16
            t1 = zero16
            t2 = zero16
            t3 = zero16
            tots = [t0, t1, t2, t3]
            for col in range(LANE):
                tots[col % 4] = tots[col % 4] + plsc.load_gather(
                    rs_v, [lanes * LANE + col])
            tot = (tots[0] + tots[1]) + (tots[2] + tots[3])
            for j in range(LANE):
                mj = (lanes == j) & valid16
                plsc.addupdate_scatter(acc_v, [cls16], tot, mask=mj)
            return carry
        lax.fori_loop(0, CH // LANE, group_body, 0)

    @pl.when(nchunks > 0)
    def _():
        fire(0, 0)

    def pair_body(gp, carry):
        for b in (0, 1):
            g = gp * 2 + b

            @pl.when(g + 1 < nchunks)
            def _():
                fire(g + 1, 1 - b)

            @pl.when(g < nchunks)
            def _():
                drain(b)
                compute_chunk(g, b)
        return carry
    lax.fori_loop(0, (nchunks + 1) // 2, pair_body, 0)

    pltpu.sync_copy(acc_v, out_hbm.at[wid])


@functools.partial(
    pl.kernel,
    out_type=jax.ShapeDtypeStruct((W, ACC), jnp.float32),
    mesh=plsc.VectorSubcoreMesh(core_axis_name="c", subcore_axis_name="s"),
    compiler_params=pltpu.CompilerParams(needs_layout_passes=False),
    scratch_types=[
        pltpu.VMEM((RPW,), jnp.int32),
        pltpu.VMEM((RPW + CH,), jnp.int32),
        pltpu.VMEM((CH, L), jnp.float32),
        pltpu.VMEM((CH, L), jnp.float32),
        pltpu.VMEM((CH, L), jnp.float32),
        pltpu.VMEM((CH, L), jnp.float32),
        pltpu.VMEM((LANE * LANE,), jnp.float32),
        pltpu.VMEM((ACC,), jnp.float32),
        pltpu.SemaphoreType.DMA,
        pltpu.SemaphoreType.DMA,
        pltpu.SemaphoreType.DMA,
        pltpu.SemaphoreType.DMA,
    ],
)
def _sc_partial_sums(s_hbm, t_hbm, tgt_hbm, out_hbm, *rest):
    _sc_body(s_hbm, t_hbm, tgt_hbm, out_hbm, *rest)


def _fin_body(parts_ref, out_ref):
    p = parts_ref[...]
    s = jnp.sum(p[:, :CPAD], axis=0, keepdims=True)
    npos = jnp.sum(p[:, CPAD:], axis=0, keepdims=True)
    denom = jnp.maximum(npos * jnp.float32(L), 1.0)
    valid = (npos > 1.0).astype(jnp.float32)
    out_ref[0, 0] = jnp.sum(s / denom * valid)


def kernel(le_student, le_teacher, targets):
    s2 = le_student.reshape(R, L)
    t2 = le_teacher.reshape(R, L)
    tgt = targets.reshape(R)
    parts = _sc_partial_sums(s2, t2, tgt)
    loss = pl.pallas_call(
        _fin_body,
        out_shape=jax.ShapeDtypeStruct((1, 1), jnp.float32),
        out_specs=pl.BlockSpec(memory_space=pltpu.SMEM),
    )(parts)
    return loss[0, 0]


# speedup vs baseline: 1.6833x; 1.1550x over previous
import functools

import jax
import jax.numpy as jnp
from jax import lax
from jax.experimental import pallas as pl
from jax.experimental.pallas import tpu as pltpu
from jax.experimental.pallas import tpu_sc as plsc

N, C, L = 1024, 80, 256
R = N * C
NUM_CORES = 2
NUM_SUBCORES = 16
W = NUM_CORES * NUM_SUBCORES
N_TC = 512
R0 = N_TC * C
RPW = (R - R0) // W
CH = 64
CPAD = 128
ACC = 2 * CPAD
LANE = 16
BN = 8

assert N_TC % 64 == 0 and RPW % LANE == 0


def _sc_body(s_hbm, t_hbm, tgt_hbm, out_hbm,
             tgt_v, idx_v, sbuf0, sbuf1, tbuf0, tbuf1, rs_v, acc_v,
             sem_s0, sem_s1, sem_t0, sem_t1):
    wid = lax.axis_index("s") * NUM_CORES + lax.axis_index("c")
    base = R0 + wid * RPW
    lanes = lax.iota(jnp.int32, LANE)
    zero16 = jnp.zeros((LANE,), jnp.float32)

    pltpu.sync_copy(tgt_hbm.at[pl.ds(base, RPW)], tgt_v)

    def zbody(i, carry):
        acc_v[pl.ds(i * LANE, LANE)] = zero16
        return carry
    lax.fori_loop(0, ACC // LANE, zbody, 0)

    def cbody(i, wptr):
        tv = tgt_v[pl.ds(i * LANE, LANE)]
        m = tv > 0
        mi = m.astype(jnp.int32)
        pos = wptr + plsc.cumsum(mi) - 1
        rowid = base + i * LANE + lanes
        plsc.store_scatter(idx_v, [pos], rowid, mask=m)
        cls = lax.rem(rowid, jnp.int32(C))
        plsc.addupdate_scatter(acc_v, [cls + CPAD], m.astype(jnp.float32))
        return wptr + jnp.sum(mi)
    count = lax.fori_loop(0, RPW // LANE, cbody, jnp.int32(0))

    basevec = jnp.full((LANE,), base, jnp.int32)
    for t in range(CH // LANE):
        plsc.store_scatter(idx_v, [count + t * LANE + lanes], basevec)

    nchunks = (count + CH - 1) // CH
    sbufs = (sbuf0, sbuf1)
    tbufs = (tbuf0, tbuf1)
    sems_s = (sem_s0, sem_s1)
    sems_t = (sem_t0, sem_t1)

    def fire(g, b):
        idxsl = idx_v.at[pl.ds(g * CH, CH)]
        pltpu.async_copy(s_hbm.at[idxsl], sbufs[b], sems_s[b])
        pltpu.async_copy(t_hbm.at[idxsl], tbufs[b], sems_t[b])

    def drain(b):
        dummy = s_hbm.at[pl.ds(0, CH)]
        pltpu.make_async_copy(dummy, sbufs[b], sems_s[b]).wait()
        pltpu.make_async_copy(dummy, tbufs[b], sems_t[b]).wait()

    def compute_chunk(g, b):
        sbuf = sbufs[b]
        tbuf = tbufs[b]

        def group_body(gg, carry):
            off = g * CH + gg * LANE
            rid16 = idx_v[pl.ds(off, LANE)]
            cls16 = lax.rem(rid16, jnp.int32(C))
            valid16 = (off + lanes) < count
            for j in range(LANE):
                rbase = gg * LANE + j
                accs = [zero16, zero16, zero16, zero16]
                for k in range(L // LANE):
                    sv = sbuf[rbase, pl.ds(k * LANE, LANE)]
                    tv2 = tbuf[rbase, pl.ds(k * LANE, LANE)]
                    d = sv - tv2
                    a = jnp.abs(d)
                    m = jnp.minimum(a, 1.0)
                    accs[k % 4] = accs[k % 4] + m * (a - 0.5 * m)
                rs_v[pl.ds(j * LANE, LANE)] = (accs[0] + accs[1]) + (accs[2] + accs[3])
            tots = [zero16, zero16, zero16, zero16]
            for col in range(LANE):
                tots[col % 4] = tots[col % 4] + plsc.load_gather(
                    rs_v, [lanes * LANE + col])
            tot = (tots[0] + tots[1]) + (tots[2] + tots[3])
            for j in range(LANE):
                mj = (lanes == j) & valid16
                plsc.addupdate_scatter(acc_v, [cls16], tot, mask=mj)
            return carry
        lax.fori_loop(0, CH // LANE, group_body, 0)

    @pl.when(nchunks > 0)
    def _():
        fire(0, 0)

    def pair_body(gp, carry):
        for b in (0, 1):
            g = gp * 2 + b

            @pl.when(g + 1 < nchunks)
            def _():
                fire(g + 1, 1 - b)

            @pl.when(g < nchunks)
            def _():
                drain(b)
                compute_chunk(g, b)
        return carry
    lax.fori_loop(0, (nchunks + 1) // 2, pair_body, 0)

    pltpu.sync_copy(acc_v, out_hbm.at[wid])


@functools.partial(
    pl.kernel,
    out_type=jax.ShapeDtypeStruct((W, ACC), jnp.float32),
    mesh=plsc.VectorSubcoreMesh(core_axis_name="c", subcore_axis_name="s"),
    compiler_params=pltpu.CompilerParams(needs_layout_passes=False),
    scratch_types=[
        pltpu.VMEM((RPW,), jnp.int32),
        pltpu.VMEM((RPW + CH,), jnp.int32),
        pltpu.VMEM((CH, L), jnp.float32),
        pltpu.VMEM((CH, L), jnp.float32),
        pltpu.VMEM((CH, L), jnp.float32),
        pltpu.VMEM((CH, L), jnp.float32),
        pltpu.VMEM((LANE * LANE,), jnp.float32),
        pltpu.VMEM((ACC,), jnp.float32),
        pltpu.SemaphoreType.DMA,
        pltpu.SemaphoreType.DMA,
        pltpu.SemaphoreType.DMA,
        pltpu.SemaphoreType.DMA,
    ],
)
def _sc_partial_sums(s_hbm, t_hbm, tgt_hbm, out_hbm, *rest):
    _sc_body(s_hbm, t_hbm, tgt_hbm, out_hbm, *rest)


def _tc_body(s_ref, t_ref, mk_ref, out_ref):
    @pl.when(pl.program_id(0) == 0)
    def _():
        out_ref[...] = jnp.zeros_like(out_ref)

    s = s_ref[...]
    t = t_ref[...]
    mk = mk_ref[...]
    d = s - t
    a = jnp.abs(d)
    m = jnp.minimum(a, 1.0)
    h = m * (a - 0.5 * m)
    psum = jnp.sum(h * mk[:, :, None], axis=(0, 2))
    pcnt = jnp.sum(mk, axis=0)
    out_ref[...] += jnp.stack([psum, pcnt])


def _fin_body(sc_ref, tc_ref, out_ref):
    p = sc_ref[...]
    s80 = jnp.sum(p[:, :CPAD], axis=0, keepdims=True)[:, :C] + tc_ref[0:1, :]
    n80 = jnp.sum(p[:, CPAD:], axis=0, keepdims=True)[:, :C] + tc_ref[1:2, :]
    denom = jnp.maximum(n80 * jnp.float32(L), 1.0)
    valid = (n80 > 1.0).astype(jnp.float32)
    out_ref[0, 0] = jnp.sum(s80 / denom * valid)


def kernel(le_student, le_teacher, targets):
    s2 = le_student.reshape(R, L)
    t2 = le_teacher.reshape(R, L)
    tgt = targets.reshape(R)
    sc_parts = _sc_partial_sums(s2, t2, tgt)

    mk = targets.astype(jnp.float32)
    tc_parts = pl.pallas_call(
        _tc_body,
        grid=(N_TC // BN,),
        in_specs=[
            pl.BlockSpec((BN, C, L), lambda i: (i, 0, 0)),
            pl.BlockSpec((BN, C, L), lambda i: (i, 0, 0)),
            pl.BlockSpec((BN, C), lambda i: (i, 0)),
        ],
        out_specs=pl.BlockSpec((2, C), lambda i: (0, 0)),
        out_shape=jax.ShapeDtypeStruct((2, C), jnp.float32),
    )(le_student, le_teacher, mk)

    loss = pl.pallas_call(
        _fin_body,
        out_shape=jax.ShapeDtypeStruct((1, 1), jnp.float32),
        out_specs=pl.BlockSpec(memory_space=pltpu.SMEM),
    )(sc_parts, tc_parts)
    return loss[0, 0]
